# baseline (speedup 1.0000x reference)
import functools

import jax
import jax.numpy as jnp
import numpy as np
from jax.experimental import pallas as pl
from jax.experimental.pallas import tpu as pltpu

_HUFFMAN_PATHS = ((0, 1), (1, 0), (0, 0, 1), (1, 1, 0))


def _body(h_ref, tw_ref, w0_ref, w1_ref, w2_ref, w3_ref, out_ref):
    h = h_ref[...]
    bm = h.shape[0]
    tw = tw_ref[...]
    n = pl.num_programs(0) * bm
    total = jnp.float32(0.0)
    for w, (w_ref, path) in enumerate(
        zip((w0_ref, w1_ref, w2_ref, w3_ref), _HUFFMAN_PATHS)
    ):
        x = jax.lax.dot_general(
            h,
            w_ref[...],
            (((1,), (1,)), ((), ())),
            preferred_element_type=jnp.float32,
        )
        ones = [j for j, b in enumerate(path) if b == 1]
        lo, hi = ones[0], ones[-1] + 1
        soft = jnp.sum(
            jnp.maximum(x, 0.0) + jnp.log1p(jnp.exp(-jnp.abs(x))),
            axis=1,
            keepdims=True,
        )
        xs = jnp.sum(x[:, lo:hi], axis=1, keepdims=True)
        per_row = (soft - xs) * (1.0 / len(path))
        sel = (tw == w).astype(jnp.float32)
        total = total + jnp.sum(sel * per_row)

    @pl.when(pl.program_id(0) == 0)
    def _():
        out_ref[0, 0] = 0.0

    out_ref[0, 0] += total / jnp.float32(n)


@functools.partial(jax.jit, static_argnames=("interpret",))
def kernel(hidden, target_words, W_0, W_1, W_2, W_3, interpret=False):
    batch, hdim = hidden.shape
    bm = 512
    grid = batch // bm

    tw2d = target_words.astype(jnp.int32).reshape(batch, 1)

    full = lambda shape: pl.BlockSpec(shape, lambda i: (0, 0))
    out = pl.pallas_call(
        _body,
        grid=(grid,),
        in_specs=[
            pl.BlockSpec((bm, hdim), lambda i: (i, 0)),
            pl.BlockSpec((bm, 1), lambda i: (i, 0)),
            full(W_0.shape),
            full(W_1.shape),
            full(W_2.shape),
            full(W_3.shape),
        ],
        out_specs=pl.BlockSpec(
            (1, 1), lambda i: (0, 0), memory_space=pltpu.SMEM
        ),
        out_shape=jax.ShapeDtypeStruct((1, 1), jnp.float32),
        interpret=interpret,
    )(hidden, tw2d, W_0, W_1, W_2, W_3)
    return out[0, 0]
